# baseline (speedup 1.0000x reference)
import functools

import jax
import jax.numpy as jnp
from jax import lax
from jax.experimental import pallas as pl
from jax.experimental.pallas import tpu as pltpu
from jax.experimental.pallas import tpu_sc as plsc

N_NODES = 10000
N_EDGES = 320000
D = 128
NC = 2
NS = 16
NW = NC * NS

PW = N_EDGES // NW
CHUNK = 72
NCHUNK = 140
PWP = NCHUNK * CHUNK
DUMMY_DST = N_NODES
NPAD = 10240
RPW = NPAD // NS
ZR = 64


def _sc_segsum(y, srcp, dstp):
    mesh = plsc.VectorSubcoreMesh(core_axis_name="c", subcore_axis_name="s")

    @functools.partial(
        pl.kernel,
        mesh=mesh,
        out_type=jax.ShapeDtypeStruct((NC, NPAD, D), jnp.float32),
        scratch_types=[
            pltpu.VMEM((PWP,), jnp.int32),
            pltpu.VMEM((PWP,), jnp.int32),
            pltpu.VMEM((CHUNK, D), jnp.float32),
            pltpu.VMEM((CHUNK, D), jnp.float32),
            pltpu.VMEM((CHUNK, D), jnp.float32),
            pltpu.VMEM_SHARED((NPAD, D), jnp.float32),
            pltpu.SemaphoreType.DMA,
            pltpu.SemaphoreType.DMA,
        ],
    )
    def k(y_hbm, src_hbm, dst_hbm, out_hbm, src_v, dst_v, rows0, rows1,
          rows2, agg_s, sem_i, sem_g):
        cid = lax.axis_index("c")
        sid = lax.axis_index("s")
        wid = cid * NS + sid
        rows = (rows0, rows1, rows2)

        pltpu.async_copy(src_hbm.at[wid], src_v, sem_i)
        pltpu.async_copy(dst_hbm.at[wid], dst_v, sem_i)

        z16 = jnp.zeros((16,), jnp.float32)

        def zrow(i, carry):
            for j in range(D // 16):
                rows0[i, pl.ds(j * 16, 16)] = z16
            return carry

        lax.fori_loop(0, ZR, zrow, 0)
        r0 = sid * RPW
        for j in range(RPW // ZR):
            pltpu.sync_copy(rows0.at[pl.ds(0, ZR)],
                            agg_s.at[pl.ds(r0 + j * ZR, ZR)])
        pltpu.make_async_copy(src_hbm.at[wid], src_v, sem_i).wait()
        pltpu.make_async_copy(dst_hbm.at[wid], dst_v, sem_i).wait()
        plsc.subcore_barrier()

        def start_gather(i, buf):
            off = pl.multiple_of(i * CHUNK, 8)
            pltpu.async_copy(y_hbm.at[src_v.at[pl.ds(off, CHUNK)]], buf,
                             sem_g)

        def wait_gather(buf):
            pltpu.make_async_copy(y_hbm.at[pl.ds(0, CHUNK)], buf,
                                  sem_g).wait()

        def scatter(i, buf):
            off = pl.multiple_of(i * CHUNK, 8)
            pltpu.sync_copy(buf, agg_s.at[dst_v.at[pl.ds(off, CHUNK)]],
                            add=True)

        for b in range(3):
            start_gather(b, rows[b])

        def body(t, carry):
            g = t * 3
            for b in range(3):
                wait_gather(rows[b])
                scatter(g + b, rows[b])
                start_gather(g + b + 3, rows[b])
            return carry

        lax.fori_loop(0, 45, body, 0)
        wait_gather(rows0)
        scatter(135, rows0)
        start_gather(138, rows0)
        wait_gather(rows1)
        scatter(136, rows1)
        start_gather(139, rows1)
        wait_gather(rows2)
        scatter(137, rows2)
        wait_gather(rows0)
        scatter(138, rows0)
        wait_gather(rows1)
        scatter(139, rows1)
        plsc.subcore_barrier()

        pltpu.sync_copy(agg_s.at[pl.ds(r0, RPW)],
                        out_hbm.at[cid].at[pl.ds(r0, RPW)])

    return k(y, srcp, dstp)


def _tc_body(agg_ref, h_ref, w_ref, b_ref, wr_ref, br_ref, o_ref):
    a = agg_ref[0] + agg_ref[1]
    t = jnp.dot(a, w_ref[...], preferred_element_type=jnp.float32) + b_ref[...]
    r = (jnp.dot(h_ref[...], wr_ref[...], preferred_element_type=jnp.float32)
         + br_ref[...])
    o_ref[...] = jnp.maximum(t, 0.0) + jnp.maximum(r, 0.0)


BLK = 400


def _tc_layer(agg2, h, wl, bl, wrl, brl):
    return pl.pallas_call(
        _tc_body,
        grid=(N_NODES // BLK,),
        in_specs=[
            pl.BlockSpec((NC, BLK, D), lambda i: (0, i, 0)),
            pl.BlockSpec((BLK, D), lambda i: (i, 0)),
            pl.BlockSpec((D, D), lambda i: (0, 0)),
            pl.BlockSpec((1, D), lambda i: (0, 0)),
            pl.BlockSpec((D, D), lambda i: (0, 0)),
            pl.BlockSpec((1, D), lambda i: (0, 0)),
        ],
        out_specs=pl.BlockSpec((BLK, D), lambda i: (i, 0)),
        out_shape=jax.ShapeDtypeStruct((N_NODES, D), jnp.float32),
    )(agg2, h, wl, bl, wrl, brl)


def kernel(x, edge_index, W, b, Wres, bres):
    src = edge_index[0].astype(jnp.int32).reshape(NW, PW)
    dst = edge_index[1].astype(jnp.int32).reshape(NW, PW)
    pad_s = jnp.zeros((NW, PWP - PW), jnp.int32)
    pad_d = jnp.full((NW, PWP - PW), DUMMY_DST, jnp.int32)
    srcp = jnp.concatenate([src, pad_s], axis=1)
    dstp = jnp.concatenate([dst, pad_d], axis=1)
    h = x
    for l in range(W.shape[0]):
        agg2 = _sc_segsum(h, srcp, dstp)
        h = _tc_layer(agg2, h, W[l], b[l].reshape(1, D),
                      Wres[l], bres[l].reshape(1, D))
    return h


# speedup vs baseline: 1.3715x; 1.3715x over previous
import functools

import jax
import jax.numpy as jnp
from jax import lax
from jax.experimental import pallas as pl
from jax.experimental.pallas import tpu as pltpu
from jax.experimental.pallas import tpu_sc as plsc

N_NODES = 10000
N_EDGES = 320000
D = 128
NC = 2
NS = 16
NW = NC * NS

PW = N_EDGES // NW
CHUNK = 48
NCHUNK = 209
PWP = NCHUNK * CHUNK
DUMMY_DST = N_NODES
NPAD = 10240
RPW = NPAD // NS
ZR = 40


def _sc_segsum(y, srcp, dstp):
    mesh = plsc.VectorSubcoreMesh(core_axis_name="c", subcore_axis_name="s")

    @functools.partial(
        pl.kernel,
        mesh=mesh,
        out_type=jax.ShapeDtypeStruct((NC, NPAD, D), jnp.float32),
        scratch_types=[
            pltpu.VMEM((PWP,), jnp.int32),
            pltpu.VMEM((PWP,), jnp.int32),
            pltpu.VMEM((CHUNK, D), jnp.float32),
            pltpu.VMEM((CHUNK, D), jnp.float32),
            pltpu.VMEM((CHUNK, D), jnp.float32),
            pltpu.VMEM((CHUNK, D), jnp.float32),
            pltpu.VMEM_SHARED((NPAD, D), jnp.float32),
            pltpu.SemaphoreType.DMA,
            pltpu.SemaphoreType.DMA,
            pltpu.SemaphoreType.DMA,
        ],
    )
    def k(y_hbm, src_hbm, dst_hbm, out_hbm, src_v, dst_v, rows0, rows1,
          rows2, rows3, agg_s, sem_i, sem_g, sem_s):
        cid = lax.axis_index("c")
        sid = lax.axis_index("s")
        wid = cid * NS + sid
        rows = (rows0, rows1, rows2, rows3)

        pltpu.async_copy(src_hbm.at[wid], src_v, sem_i)
        pltpu.async_copy(dst_hbm.at[wid], dst_v, sem_i)

        z16 = jnp.zeros((16,), jnp.float32)

        def zrow(i, carry):
            for j in range(D // 16):
                rows0[i, pl.ds(j * 16, 16)] = z16
            return carry

        lax.fori_loop(0, ZR, zrow, 0)
        r0 = sid * RPW
        for j in range(RPW // ZR):
            pltpu.sync_copy(rows0.at[pl.ds(0, ZR)],
                            agg_s.at[pl.ds(r0 + j * ZR, ZR)])
        pltpu.make_async_copy(src_hbm.at[wid], src_v, sem_i).wait()
        pltpu.make_async_copy(dst_hbm.at[wid], dst_v, sem_i).wait()
        plsc.subcore_barrier()

        def start_gather(i, buf):
            off = pl.multiple_of(i * CHUNK, 8)
            pltpu.async_copy(y_hbm.at[src_v.at[pl.ds(off, CHUNK)]], buf,
                             sem_g)

        def wait_gather(buf):
            pltpu.make_async_copy(y_hbm.at[pl.ds(0, CHUNK)], buf,
                                  sem_g).wait()

        def start_scatter(i, buf):
            off = pl.multiple_of(i * CHUNK, 8)
            pltpu.async_copy(buf, agg_s.at[dst_v.at[pl.ds(off, CHUNK)]],
                             sem_s, add=True)

        def wait_scatter(buf):
            pltpu.make_async_copy(y_hbm.at[pl.ds(0, CHUNK)], buf,
                                  sem_s).wait()

        start_gather(0, rows[0])
        start_gather(1, rows[1])
        start_gather(2, rows[2])
        wait_gather(rows[0])
        start_scatter(0, rows[0])
        start_gather(3, rows[3])

        def body(t, carry):
            g = t * 4 + 1
            for b4 in range(4):
                b = (b4 + 1) % 4
                wait_gather(rows[b])
                start_scatter(g + b4, rows[b])
                wait_scatter(rows[(b + 3) % 4])
                start_gather(g + b4 + 3, rows[(b + 3) % 4])
            return carry

        lax.fori_loop(0, 51, body, 0)
        wait_gather(rows[1])
        start_scatter(205, rows[1])
        wait_scatter(rows[0])
        start_gather(208, rows[0])
        wait_gather(rows[2])
        start_scatter(206, rows[2])
        wait_scatter(rows[1])
        wait_gather(rows[3])
        start_scatter(207, rows[3])
        wait_scatter(rows[2])
        wait_gather(rows[0])
        start_scatter(208, rows[0])
        wait_scatter(rows[3])
        wait_scatter(rows[0])
        plsc.subcore_barrier()

        pltpu.sync_copy(agg_s.at[pl.ds(r0, RPW)],
                        out_hbm.at[cid].at[pl.ds(r0, RPW)])

    return k(y, srcp, dstp)


def _tc_body(agg_ref, h_ref, w_ref, b_ref, wr_ref, br_ref, o_ref):
    a = agg_ref[0] + agg_ref[1]
    t = jnp.dot(a, w_ref[...], preferred_element_type=jnp.float32) + b_ref[...]
    r = (jnp.dot(h_ref[...], wr_ref[...], preferred_element_type=jnp.float32)
         + br_ref[...])
    o_ref[...] = jnp.maximum(t, 0.0) + jnp.maximum(r, 0.0)


BLK = 400


def _tc_layer(agg2, h, wl, bl, wrl, brl):
    return pl.pallas_call(
        _tc_body,
        grid=(N_NODES // BLK,),
        in_specs=[
            pl.BlockSpec((NC, BLK, D), lambda i: (0, i, 0)),
            pl.BlockSpec((BLK, D), lambda i: (i, 0)),
            pl.BlockSpec((D, D), lambda i: (0, 0)),
            pl.BlockSpec((1, D), lambda i: (0, 0)),
            pl.BlockSpec((D, D), lambda i: (0, 0)),
            pl.BlockSpec((1, D), lambda i: (0, 0)),
        ],
        out_specs=pl.BlockSpec((BLK, D), lambda i: (i, 0)),
        out_shape=jax.ShapeDtypeStruct((N_NODES, D), jnp.float32),
    )(agg2, h, wl, bl, wrl, brl)


def kernel(x, edge_index, W, b, Wres, bres):
    src = edge_index[0].astype(jnp.int32).reshape(NW, PW)
    dst = edge_index[1].astype(jnp.int32).reshape(NW, PW)
    pad_s = jnp.zeros((NW, PWP - PW), jnp.int32)
    pad_d = jnp.full((NW, PWP - PW), DUMMY_DST, jnp.int32)
    srcp = jnp.concatenate([src, pad_s], axis=1)
    dstp = jnp.concatenate([dst, pad_d], axis=1)
    h = x
    for l in range(W.shape[0]):
        agg2 = _sc_segsum(h, srcp, dstp)
        h = _tc_layer(agg2, h, W[l], b[l].reshape(1, D),
                      Wres[l], bres[l].reshape(1, D))
    return h
